# baseline (speedup 1.0000x reference)
import functools

import jax
import jax.numpy as jnp
from jax import lax
from jax.experimental import pallas as pl
from jax.experimental.pallas import tpu as pltpu
from jax.experimental.pallas import tpu_sc as plsc


def _make_probe(B, NC, NW, b_per_w):
    mesh = plsc.VectorSubcoreMesh(core_axis_name="c", subcore_axis_name="s")

    @functools.partial(
        pl.kernel,
        mesh=mesh,
        out_type=jax.ShapeDtypeStruct((B,), jnp.int32),
        scratch_types=[
            pltpu.VMEM((b_per_w,), jnp.int32),
        ],
    )
    def probe(word_hbm, out_hbm, idx_v):
        wid = lax.axis_index("s") * NC + lax.axis_index("c")
        base = wid * b_per_w
        pltpu.sync_copy(word_hbm.at[pl.ds(base, b_per_w)], idx_v)
        pltpu.sync_copy(idx_v, out_hbm.at[pl.ds(base, b_per_w)])

    return probe


def kernel(word, table):
    (B,) = word.shape
    _, D = table.shape
    info = plsc.get_sparse_core_info()
    NC, NS = info.num_cores, info.num_subcores
    NW = NC * NS
    b_per_w = B // NW
    probe = _make_probe(B, NC, NW, b_per_w)
    out = probe(word)
    return out.astype(jnp.float32).reshape(1, 1, -1)


# speedup vs baseline: 3.1908x; 3.1908x over previous
import functools

import jax
import jax.numpy as jnp
from jax import lax
from jax.experimental import pallas as pl
from jax.experimental.pallas import tpu as pltpu


def _emb_body(B, D, word_smem, table_hbm, out_vmem, sem):
    UNROLL = 8

    def issue(j, _):
        for u in range(UNROLL):
            i = j * UNROLL + u
            idx = word_smem[i]
            pltpu.make_async_copy(
                table_hbm.at[pl.ds(idx, 1), :],
                out_vmem.at[pl.ds(i, 1), :],
                sem,
            ).start()
        return 0

    lax.fori_loop(0, B // UNROLL, issue, 0)
    pltpu.make_async_copy(table_hbm.at[pl.ds(0, B), :], out_vmem, sem).wait()


def kernel(word, table):
    (B,) = word.shape
    _, D = table.shape

    out = pl.pallas_call(
        functools.partial(_emb_body, B, D),
        in_specs=[
            pl.BlockSpec(memory_space=pltpu.SMEM),
            pl.BlockSpec(memory_space=pl.ANY),
        ],
        out_specs=pl.BlockSpec(memory_space=pltpu.VMEM),
        out_shape=jax.ShapeDtypeStruct((B, D), jnp.float32),
        scratch_shapes=[pltpu.SemaphoreType.DMA],
    )(word, table)
    return out.reshape(1, 1, -1)
